# baseline (speedup 1.0000x reference)
import functools

import jax
import jax.numpy as jnp
from jax import lax
from jax.experimental import pallas as pl
from jax.experimental.pallas import tpu as pltpu
from jax.experimental.pallas import tpu_sc as plsc

L = 16
NW = 32
CHUNK = 32


def _gather_max_kernel(n_pos, bc, bc_pad, k, table_hbm, idx_hbm, out_hbm,
                       idx_v, rows0, rows1, out0, out1,
                       gsem0, gsem1, osem0, osem1):
    nc = 2
    wid = lax.axis_index("s") * nc + lax.axis_index("c")
    n_per = n_pos // NW
    cw = CHUNK * k
    n_chunks = n_per // CHUNK
    base = wid * n_per

    rows = (rows0, rows1)
    outs = (out0, out1)
    gsems = (gsem0, gsem1)
    osems = (osem0, osem1)

    pltpu.sync_copy(idx_hbm.at[pl.ds(base * k, n_per * k)], idx_v)

    def gather_start(c, b):
        pltpu.async_copy(table_hbm.at[idx_v.at[pl.ds(c * cw, cw)]],
                         rows[b], gsems[b])

    def gather_wait(c, b):
        pltpu.make_async_copy(table_hbm.at[idx_v.at[pl.ds(c * cw, cw)]],
                              rows[b], gsems[b]).wait()

    for b in range(2):
        gather_start(b, b)

    def half(it, b):
        c = 2 * it + b
        gather_wait(c, b)

        @pl.when(it > 0)
        def _():
            pltpu.make_async_copy(
                outs[b], out_hbm.at[pl.ds(base, CHUNK)], osems[b]).wait()

        def pos_body(i, _):
            for j in range(bc // L):
                sl = pl.ds(j * L, L)
                m0 = jnp.maximum(rows[b][k * i, sl], rows[b][k * i + 1, sl])
                m1 = jnp.maximum(rows[b][k * i + 2, sl], rows[b][k * i + 3, sl])
                outs[b][i, sl] = jnp.maximum(m0, m1)
            return 0

        lax.fori_loop(0, CHUNK, pos_body, 0)

        pltpu.async_copy(outs[b],
                         out_hbm.at[pl.ds(base + c * CHUNK, CHUNK)], osems[b])
        cn = jnp.minimum(c + 2, n_chunks - 1)
        gather_start(cn, b)

    def step(it, _):
        half(it, 0)
        half(it, 1)
        return 0

    lax.fori_loop(0, n_chunks // 2, step, 0)

    for b in range(2):
        gather_wait(n_chunks - 1, b)
        pltpu.make_async_copy(
            outs[b], out_hbm.at[pl.ds(base, CHUNK)], osems[b]).wait()


def _gather_max(table, idx, n_pos, bc, bc_pad, k):
    mesh = plsc.VectorSubcoreMesh(core_axis_name="c", subcore_axis_name="s")
    n_per = n_pos // NW
    kern = pl.kernel(
        functools.partial(_gather_max_kernel, n_pos, bc, bc_pad, k),
        out_type=jax.ShapeDtypeStruct((n_pos, bc), jnp.float32),
        mesh=mesh,
        scratch_types=[
            pltpu.VMEM((n_per * k,), jnp.int32),
            pltpu.VMEM((CHUNK * k, bc_pad), jnp.float32),
            pltpu.VMEM((CHUNK * k, bc_pad), jnp.float32),
            pltpu.VMEM((CHUNK, bc), jnp.float32),
            pltpu.VMEM((CHUNK, bc), jnp.float32),
            pltpu.SemaphoreType.DMA,
            pltpu.SemaphoreType.DMA,
            pltpu.SemaphoreType.DMA,
            pltpu.SemaphoreType.DMA,
        ],
        compiler_params=pltpu.CompilerParams(use_tc_tiling_on_sc=True),
    )
    return kern(table, idx)


def _transpose_pad_kernel(bc, bcp, hblk, w, in_ref, out_ref):
    for hh in range(hblk):
        s = in_ref[:, hh, :]
        out_ref[hh, :, :bc] = jnp.swapaxes(s, 0, 1)
        out_ref[hh, :, bc:] = jnp.zeros((w, bcp - bc), jnp.float32)


def _transpose_pad(x, bc, h, w, bcp):
    hblk = 8
    out = pl.pallas_call(
        functools.partial(_transpose_pad_kernel, bc, bcp, hblk, w),
        grid=(h // hblk,),
        in_specs=[pl.BlockSpec((bc, hblk, w), lambda g: (0, g, 0))],
        out_specs=pl.BlockSpec((hblk, w, bcp), lambda g: (g, 0, 0)),
        out_shape=jax.ShapeDtypeStruct((h, w, bcp), jnp.float32),
    )(x.reshape(bc, h, w))
    return out.reshape(h * w, bcp)


def _out_t_kernel(bc, hblk, ow, in_ref, out_ref):
    for hh in range(hblk):
        s = in_ref[hh]
        out_ref[:, hh, :] = jnp.swapaxes(s, 0, 1)


def _out_transpose(p, bc, oh, ow):
    hblk = 8
    return pl.pallas_call(
        functools.partial(_out_t_kernel, bc, hblk, ow),
        grid=(oh // hblk,),
        in_specs=[pl.BlockSpec((hblk, ow, bc), lambda g: (g, 0, 0))],
        out_specs=pl.BlockSpec((bc, hblk, ow), lambda g: (0, g, 0)),
        out_shape=jax.ShapeDtypeStruct((bc, oh, ow), jnp.float32),
    )(p.reshape(oh, ow, bc))


def kernel(x, sample_map):
    B, C, H, W = x.shape
    OH, OW, K, _ = sample_map.shape
    BC = B * C
    N = OH * OW

    coords = jnp.round(sample_map).astype(jnp.int32)
    cx = jnp.clip(coords[..., 0], 0, W - 1)
    cy = jnp.clip(coords[..., 1], 0, H - 1)
    flat = (cy * W + cx).reshape(N * K)

    del BC
    CP = 128
    parts = []
    for b in range(B):
        table_b = _transpose_pad(x[b], C, H, W, CP)
        pooled_b = _gather_max(table_b, flat, N, C, CP, K)
        parts.append(_out_transpose(pooled_b, C, OH, OW))
    return jnp.stack(parts).reshape(B, C, OH, OW)


# speedup vs baseline: 1.3757x; 1.3757x over previous
import functools

import jax
import jax.numpy as jnp
from jax import lax
from jax.experimental import pallas as pl
from jax.experimental.pallas import tpu as pltpu
from jax.experimental.pallas import tpu_sc as plsc

L = 16
NW = 32
CHUNK = 32


def _gather_max_kernel(n_pos, bc, bc_pad, k, table_hbm, idx_hbm, out_hbm,
                       idx_v, rows0, rows1, out0, out1,
                       gsem0, gsem1, osem0, osem1):
    nc = 2
    wid = lax.axis_index("s") * nc + lax.axis_index("c")
    n_per = n_pos // NW
    cw = CHUNK * k
    n_chunks = n_per // CHUNK
    base = wid * n_per

    rows = (rows0, rows1)
    outs = (out0, out1)
    gsems = (gsem0, gsem1)
    osems = (osem0, osem1)

    pltpu.sync_copy(idx_hbm.at[pl.ds(base * k, n_per * k)], idx_v)

    def gather_start(c, b):
        pltpu.async_copy(table_hbm.at[idx_v.at[pl.ds(c * cw, cw)]],
                         rows[b], gsems[b])

    def gather_wait(c, b):
        pltpu.make_async_copy(table_hbm.at[idx_v.at[pl.ds(c * cw, cw)]],
                              rows[b], gsems[b]).wait()

    for b in range(2):
        gather_start(b, b)

    def half(it, b):
        c = 2 * it + b
        gather_wait(c, b)

        @pl.when(it > 0)
        def _():
            pltpu.make_async_copy(
                outs[b], out_hbm.at[pl.ds(base, CHUNK)], osems[b]).wait()

        def pos_body(i, _):
            for j in range(bc // L):
                sl = pl.ds(j * L, L)
                m0 = jnp.maximum(rows[b][k * i, sl], rows[b][k * i + 1, sl])
                m1 = jnp.maximum(rows[b][k * i + 2, sl], rows[b][k * i + 3, sl])
                outs[b][i, sl] = jnp.maximum(m0, m1)
            return 0

        lax.fori_loop(0, CHUNK, pos_body, 0)

        pltpu.async_copy(outs[b],
                         out_hbm.at[pl.ds(base + c * CHUNK, CHUNK)], osems[b])
        cn = jnp.minimum(c + 2, n_chunks - 1)
        gather_start(cn, b)

    def step(it, _):
        half(it, 0)
        half(it, 1)
        return 0

    lax.fori_loop(0, n_chunks // 2, step, 0)

    for b in range(2):
        gather_wait(n_chunks - 1, b)
        pltpu.make_async_copy(
            outs[b], out_hbm.at[pl.ds(base, CHUNK)], osems[b]).wait()


def _gather_max(table, idx, n_pos, bc, bc_pad, k):
    mesh = plsc.VectorSubcoreMesh(core_axis_name="c", subcore_axis_name="s")
    n_per = n_pos // NW
    kern = pl.kernel(
        functools.partial(_gather_max_kernel, n_pos, bc, bc_pad, k),
        out_type=jax.ShapeDtypeStruct((n_pos, bc), jnp.float32),
        mesh=mesh,
        scratch_types=[
            pltpu.VMEM((n_per * k,), jnp.int32),
            pltpu.VMEM((CHUNK * k, bc_pad), jnp.float32),
            pltpu.VMEM((CHUNK * k, bc_pad), jnp.float32),
            pltpu.VMEM((CHUNK, bc), jnp.float32),
            pltpu.VMEM((CHUNK, bc), jnp.float32),
            pltpu.SemaphoreType.DMA,
            pltpu.SemaphoreType.DMA,
            pltpu.SemaphoreType.DMA,
            pltpu.SemaphoreType.DMA,
        ],
        compiler_params=pltpu.CompilerParams(use_tc_tiling_on_sc=True),
    )
    return kern(table, idx)


def _transpose_pad_kernel(bc, bcp, hblk, w, in_ref, out_ref):
    for hh in range(hblk):
        s = in_ref[:, hh, :]
        out_ref[hh, :, :bc] = jnp.swapaxes(s, 0, 1)
        out_ref[hh, :, bc:] = jnp.zeros((w, bcp - bc), jnp.float32)


def _transpose_pad(x, bc, h, w, bcp):
    hblk = 16
    out = pl.pallas_call(
        functools.partial(_transpose_pad_kernel, bc, bcp, hblk, w),
        grid=(h // hblk,),
        in_specs=[pl.BlockSpec((bc, hblk, w), lambda g: (0, g, 0))],
        out_specs=pl.BlockSpec((hblk, w, bcp), lambda g: (g, 0, 0)),
        out_shape=jax.ShapeDtypeStruct((h, w, bcp), jnp.float32),
    )(x.reshape(bc, h, w))
    return out.reshape(h * w, bcp)


def _out_t_kernel(bc, hblk, ow, in_ref, out_ref):
    for hh in range(hblk):
        s = in_ref[hh]
        out_ref[:, hh, :] = jnp.swapaxes(s, 0, 1)


def _out_transpose(p, bc, oh, ow):
    hblk = 32
    return pl.pallas_call(
        functools.partial(_out_t_kernel, bc, hblk, ow),
        grid=(oh // hblk,),
        in_specs=[pl.BlockSpec((hblk, ow, bc), lambda g: (g, 0, 0))],
        out_specs=pl.BlockSpec((bc, hblk, ow), lambda g: (0, g, 0)),
        out_shape=jax.ShapeDtypeStruct((bc, oh, ow), jnp.float32),
    )(p.reshape(oh, ow, bc))


def kernel(x, sample_map):
    B, C, H, W = x.shape
    OH, OW, K, _ = sample_map.shape
    BC = B * C
    N = OH * OW

    coords = jnp.round(sample_map).astype(jnp.int32)
    cx = jnp.clip(coords[..., 0], 0, W - 1)
    cy = jnp.clip(coords[..., 1], 0, H - 1)
    flat = (cy * W + cx).reshape(N * K)

    BCP = 256
    table = _transpose_pad(x, BC, H, W, BCP)
    pooled_t = _gather_max(table, flat, N, BC, BCP, K)
    return _out_transpose(pooled_t, BC, OH, OW).reshape(B, C, OH, OW)
